# baseline (speedup 1.0000x reference)
import functools

import jax
import jax.numpy as jnp
from jax import lax
from jax.experimental import pallas as pl
from jax.experimental.pallas import tpu as pltpu
from jax.experimental.pallas import tpu_sc as plsc

B = 64
S = 1024
D = 768
N = B * S
L = 16

NUM_WORKERS = 32
PER_W = N // NUM_WORKERS
REPS = PER_W // S
C = 64
NPCHUNKS = S // C


def _emb_body(x_hbm, tok_hbm, pos_hbm, out_hbm, idx_v, rows_v, pos_v, sem):
    wid = lax.axis_index("s") * 2 + lax.axis_index("c")
    base = wid * PER_W

    def pchunk(c, carry):
        p0 = c * C
        pltpu.sync_copy(pos_hbm.at[pl.ds(p0, C)], pos_v)
        for r in range(REPS):
            start = base + r * S + p0
            pltpu.sync_copy(x_hbm.at[pl.ds(start, C)], idx_v)
            pltpu.async_copy(tok_hbm.at[idx_v], rows_v, sem).wait()

            def addrow(j, carry2):
                for k in range(D // L):
                    sl = pl.ds(k * L, L)
                    plsc.addupdate(rows_v.at[j, sl], pos_v[j, sl])
                return carry2

            lax.fori_loop(0, C, addrow, 0)
            pltpu.sync_copy(rows_v, out_hbm.at[pl.ds(start, C)])
        return carry

    lax.fori_loop(0, NPCHUNKS, pchunk, 0)


@jax.jit
def _emb(x_flat, token_emb, pos2d):
    mesh = plsc.VectorSubcoreMesh(core_axis_name="c", subcore_axis_name="s")
    f = functools.partial(
        pl.kernel,
        out_type=jax.ShapeDtypeStruct((N, D), jnp.float32),
        mesh=mesh,
        scratch_types=[
            pltpu.VMEM((C,), jnp.int32),
            pltpu.VMEM((C, D), jnp.float32),
            pltpu.VMEM((C, D), jnp.float32),
            pltpu.SemaphoreType.DMA,
        ],
    )(_emb_body)
    return f(x_flat, token_emb, pos2d)


def kernel(x, token_emb, pos_emb):
    x_flat = x.reshape(N)
    pos2d = pos_emb.reshape(S, D)
    out = _emb(x_flat, token_emb, pos2d)
    return out.reshape(B, S, D)
